# baseline (speedup 1.0000x reference)
import functools

import jax
import jax.numpy as jnp
from jax import lax
from jax.experimental import pallas as pl
from jax.experimental.pallas import tpu as pltpu
from jax.experimental.pallas import tpu_sc as plsc

B = 16384
D = 64
V = 1000000
L = 16
NC = 2
NS = 16
NW = NC * NS

W = 128
NB = 7812
TAIL = NB * W
ITERS = 245

CH = 128
NCHUNKS = B // CH
CPW = NCHUNKS // NW

_MESH = dict(core_axis_name="c", subcore_axis_name="s")


def _sc_relayout(wt, wt_tail):
    @functools.partial(
        pl.kernel,
        mesh=plsc.VectorSubcoreMesh(**_MESH),
        out_type=jax.ShapeDtypeStruct((V * D,), jnp.float32),
        scratch_types=[
            pltpu.VMEM((2, D, W), jnp.float32),
            pltpu.VMEM((2, W * D), jnp.float32),
            pltpu.SemaphoreType.DMA,
            pltpu.SemaphoreType.DMA,
        ],
        compiler_params=pltpu.CompilerParams(
            use_tc_tiling_on_sc=True, needs_layout_passes=False
        ),
    )
    def k1(wt_hbm, tail_hbm, out_hbm, stage_v, buft_v, ssem, wsem):
        wid = lax.axis_index("s") * NC + lax.axis_index("c")
        nv = 244 + jnp.where(wid < 4, 1, 0)

        def col_start(b):
            t = b * NW + wid
            return pl.multiple_of(t * W, W)

        def stage_in(b, buf):
            pltpu.async_copy(
                wt_hbm.at[:, pl.ds(col_start(b), W)],
                stage_v.at[buf],
                ssem,
            )

        def wait_stage(buf):
            pltpu.make_async_copy(
                wt_hbm.at[:, pl.ds(0, W)],
                stage_v.at[buf],
                ssem,
            ).wait()

        def wait_write(buf):
            pltpu.make_async_copy(
                buft_v.at[buf], out_hbm.at[pl.ds(0, W * D)], wsem
            ).wait()

        cvecs = [lax.iota(jnp.int32, L) + (g * L) for g in range(4)]

        stage_in(0, 0)

        def body(b, carry):
            cur = lax.rem(b, 2)

            @pl.when(b < nv)
            def _valid():
                wait_stage(cur)

                @pl.when(b + 1 < nv)
                def _prefetch():
                    stage_in(b + 1, 1 - cur)

                @pl.when(b >= 2)
                def _drain_write():
                    wait_write(cur)

                curvec = jnp.full((L,), 0, jnp.int32) + cur

                def shuffle(i, c2):
                    ivec = jnp.full((L,), 0, jnp.int32) + i
                    base = i * D
                    for g in range(4):
                        vals = plsc.load_gather(
                            stage_v, [curvec, cvecs[g], ivec]
                        )
                        buft_v[cur, pl.ds(base + g * L, L)] = vals
                    return c2

                lax.fori_loop(0, W, shuffle, 0)

                pltpu.async_copy(
                    buft_v.at[cur],
                    out_hbm.at[pl.ds(col_start(b) * D, W * D)],
                    wsem,
                )

            return carry

        lax.fori_loop(0, ITERS, body, 0)
        wait_write(0)
        wait_write(1)

        @pl.when(wid == 0)
        def _tail():
            pltpu.sync_copy(tail_hbm, stage_v.at[0])
            zvec = jnp.zeros((L,), jnp.int32)

            def shuffle_tail(i, c2):
                ivec = jnp.full((L,), 0, jnp.int32) + i
                base = i * D
                for g in range(4):
                    vals = plsc.load_gather(stage_v, [zvec, cvecs[g], ivec])
                    buft_v[0, pl.ds(base + g * L, L)] = vals
                return c2

            lax.fori_loop(0, W, shuffle_tail, 0)
            pltpu.sync_copy(
                buft_v.at[0],
                out_hbm.at[pl.ds((V - W) * D, W * D)],
            )

    return k1(wt, wt_tail)


def _sc_gather(idx2d, w2d):
    @functools.partial(
        pl.kernel,
        mesh=plsc.VectorSubcoreMesh(**_MESH),
        out_type=jax.ShapeDtypeStruct((NCHUNKS, CH, D), jnp.float32),
        scratch_types=[
            pltpu.VMEM((CPW, CH), jnp.int32),
            pltpu.VMEM((CPW, CH, D), jnp.float32),
            pltpu.SemaphoreType.DMA,
        ],
        compiler_params=pltpu.CompilerParams(use_tc_tiling_on_sc=False),
    )
    def k2(idx_hbm, table_hbm, out_hbm, idx_v, rows_v, sem):
        wid = lax.axis_index("s") * NC + lax.axis_index("c")
        base = wid * CPW
        pltpu.sync_copy(idx_hbm.at[pl.ds(base, CPW)], idx_v)
        descs = []
        for j in range(CPW):
            descs.append(
                pltpu.async_copy(table_hbm.at[idx_v.at[j]], rows_v.at[j], sem)
            )
        for d in descs:
            d.wait()
        pltpu.sync_copy(rows_v, out_hbm.at[pl.ds(base, CPW)])

    return k2(idx2d, w2d)


def kernel(idx, weight):
    idx2d = idx.astype(jnp.int32).reshape(NCHUNKS, CH)
    wflat = _sc_relayout(weight.T, weight[V - W :, :].T)
    w2d = wflat.reshape(V, D)
    out = _sc_gather(idx2d, w2d)
    return out.reshape(B, D)


# speedup vs baseline: 1.2145x; 1.2145x over previous
import functools

import jax
import jax.numpy as jnp
from jax import lax
from jax.experimental import pallas as pl
from jax.experimental.pallas import tpu as pltpu
from jax.experimental.pallas import tpu_sc as plsc

B = 16384
D = 64
V = 1000000
L = 16
NC = 2
NS = 16
NW = NC * NS

W = 128
NB = 7812
TAIL = NB * W
ITERS = 245

CH = 128
NCHUNKS = B // CH
CPW = NCHUNKS // NW

_MESH = dict(core_axis_name="c", subcore_axis_name="s")


def _sc_relayout(wt, wt_tail):
    @functools.partial(
        pl.kernel,
        mesh=plsc.VectorSubcoreMesh(**_MESH),
        out_type=jax.ShapeDtypeStruct((V * D,), jnp.float32),
        scratch_types=[
            pltpu.VMEM((2, D, W), jnp.float32),
            pltpu.VMEM((2, W * D), jnp.float32),
            pltpu.SemaphoreType.DMA,
            pltpu.SemaphoreType.DMA,
        ],
        compiler_params=pltpu.CompilerParams(
            use_tc_tiling_on_sc=True, needs_layout_passes=False
        ),
    )
    def k1(wt_hbm, tail_hbm, out_hbm, stage_v, buft_v, ssem, wsem):
        wid = lax.axis_index("s") * NC + lax.axis_index("c")
        nv = 244 + jnp.where(wid < 4, 1, 0)

        def col_start(b):
            t = b * NW + wid
            return pl.multiple_of(t * W, W)

        def stage_in(b, buf):
            pltpu.async_copy(
                wt_hbm.at[:, pl.ds(col_start(b), W)],
                stage_v.at[buf],
                ssem,
            )

        def wait_stage(buf):
            pltpu.make_async_copy(
                wt_hbm.at[:, pl.ds(0, W)],
                stage_v.at[buf],
                ssem,
            ).wait()

        def wait_write(buf):
            pltpu.make_async_copy(
                buft_v.at[buf], out_hbm.at[pl.ds(0, W * D)], wsem
            ).wait()

        posvecs_w = [(lax.iota(jnp.int32, L) + (g * L)) * D for g in range(W // L)]

        stage_in(0, 0)

        def body(b, carry):
            cur = lax.rem(b, 2)

            @pl.when(b < nv)
            def _valid():
                wait_stage(cur)

                @pl.when(b + 1 < nv)
                def _prefetch():
                    stage_in(b + 1, 1 - cur)

                @pl.when(b >= 2)
                def _drain_write():
                    wait_write(cur)

                curvec = jnp.full((L,), 0, jnp.int32) + cur

                def shuffle(c, c2):
                    cvec = jnp.full((L,), 0, jnp.int32) + c
                    for g in range(W // L):
                        v = stage_v[cur, c, pl.ds(g * L, L)]
                        plsc.store_scatter(
                            buft_v, [curvec, posvecs_w[g] + cvec], v
                        )
                    return c2

                lax.fori_loop(0, D, shuffle, 0)

                pltpu.async_copy(
                    buft_v.at[cur],
                    out_hbm.at[pl.ds(col_start(b) * D, W * D)],
                    wsem,
                )

            return carry

        lax.fori_loop(0, ITERS, body, 0)
        wait_write(0)
        wait_write(1)

        @pl.when(wid == 0)
        def _tail():
            pltpu.sync_copy(tail_hbm, stage_v.at[0])
            zvec = jnp.zeros((L,), jnp.int32)

            def shuffle_tail(c, c2):
                cvec = jnp.full((L,), 0, jnp.int32) + c
                for g in range(W // L):
                    v = stage_v[0, c, pl.ds(g * L, L)]
                    plsc.store_scatter(buft_v, [zvec, posvecs_w[g] + cvec], v)
                return c2

            lax.fori_loop(0, D, shuffle_tail, 0)
            pltpu.sync_copy(
                buft_v.at[0],
                out_hbm.at[pl.ds((V - W) * D, W * D)],
            )

    return k1(wt, wt_tail)


def _sc_gather(idx2d, w2d):
    @functools.partial(
        pl.kernel,
        mesh=plsc.VectorSubcoreMesh(**_MESH),
        out_type=jax.ShapeDtypeStruct((NCHUNKS, CH, D), jnp.float32),
        scratch_types=[
            pltpu.VMEM((CPW, CH), jnp.int32),
            pltpu.VMEM((CPW, CH, D), jnp.float32),
            pltpu.SemaphoreType.DMA,
        ],
        compiler_params=pltpu.CompilerParams(use_tc_tiling_on_sc=False),
    )
    def k2(idx_hbm, table_hbm, out_hbm, idx_v, rows_v, sem):
        wid = lax.axis_index("s") * NC + lax.axis_index("c")
        base = wid * CPW
        pltpu.sync_copy(idx_hbm.at[pl.ds(base, CPW)], idx_v)
        descs = []
        for j in range(CPW):
            descs.append(
                pltpu.async_copy(table_hbm.at[idx_v.at[j]], rows_v.at[j], sem)
            )
        for d in descs:
            d.wait()
        pltpu.sync_copy(rows_v, out_hbm.at[pl.ds(base, CPW)])

    return k2(idx2d, w2d)


def kernel(idx, weight):
    idx2d = idx.astype(jnp.int32).reshape(NCHUNKS, CH)
    wflat = _sc_relayout(weight.T, weight[V - W :, :].T)
    w2d = wflat.reshape(V, D)
    out = _sc_gather(idx2d, w2d)
    return out.reshape(B, D)


# speedup vs baseline: 2.5619x; 2.1094x over previous
import functools

import jax
import jax.numpy as jnp
from jax import lax
from jax.experimental import pallas as pl
from jax.experimental.pallas import tpu as pltpu
from jax.experimental.pallas import tpu_sc as plsc

B = 16384
D = 64
CHUNK = 128
NC = 2
NS = 16
NW = NC * NS
NCHUNKS = B // CHUNK
CPW = NCHUNKS // NW


def _sc_gather(idx2d, weight):
    mesh = plsc.VectorSubcoreMesh(core_axis_name="c", subcore_axis_name="s")

    @functools.partial(
        pl.kernel,
        mesh=mesh,
        out_type=jax.ShapeDtypeStruct((NCHUNKS, CHUNK, D), jnp.float32),
        scratch_types=[
            pltpu.VMEM((CPW, CHUNK), jnp.int32),
            pltpu.VMEM((CPW, CHUNK, D), jnp.float32),
            pltpu.SemaphoreType.DMA,
        ],
        compiler_params=pltpu.CompilerParams(use_tc_tiling_on_sc=False),
    )
    def k(idx_hbm, table_hbm, out_hbm, idx_v, rows_v, sem):
        wid = lax.axis_index("s") * NC + lax.axis_index("c")
        base = wid * CPW
        pltpu.sync_copy(idx_hbm.at[pl.ds(base, CPW)], idx_v)
        descs = []
        for j in range(CPW):
            descs.append(
                pltpu.async_copy(table_hbm.at[idx_v.at[j]], rows_v.at[j], sem)
            )
        for d in descs:
            d.wait()
        pltpu.sync_copy(rows_v, out_hbm.at[pl.ds(base, CPW)])

    return k(idx2d, weight)


def kernel(idx, weight):
    idx2d = idx.astype(jnp.int32).reshape(NCHUNKS, CHUNK)
    out = _sc_gather(idx2d, weight)
    return out.reshape(B, D)
